# baseline (speedup 1.0000x reference)
import functools

import jax
import jax.numpy as jnp
from jax import lax
from jax.experimental import pallas as pl
from jax.experimental.pallas import tpu as pltpu
from jax.experimental.pallas import tpu_sc as plsc

TOPK = 2
NEXP = 8
BLOCK = 1024

SC_TOKENS = 2048


def _gate_kernel(x_ref, w_ref, idx_ref, wgt_ref):
    x = x_ref[...]
    w = w_ref[...]
    logits = jax.lax.dot_general(
        w, x, (((1,), (1,)), ((), ())), preferred_element_type=jnp.float32
    )
    T = logits.shape[1]
    iota = jax.lax.broadcasted_iota(jnp.int32, (NEXP, T), 0)
    m1 = jnp.max(logits, axis=0, keepdims=True)
    idx1 = jnp.min(jnp.where(logits == m1, iota, NEXP), axis=0, keepdims=True)
    masked = jnp.where(iota == idx1, -jnp.inf, logits)
    m2 = jnp.max(masked, axis=0, keepdims=True)
    idx2 = jnp.min(jnp.where(masked == m2, iota, NEXP), axis=0, keepdims=True)
    z = jnp.sum(jnp.exp(logits - m1), axis=0, keepdims=True)
    w1 = 1.0 / z
    w2 = jnp.exp(m2 - m1) / z
    idx_ref[...] = jnp.concatenate([idx1, idx2], axis=0)
    wgt_ref[...] = jnp.concatenate([w1, w2], axis=0)


def _tc_gate(x, weight):
    n, h = x.shape
    grid = (n // BLOCK,)
    return pl.pallas_call(
        _gate_kernel,
        grid=grid,
        in_specs=[
            pl.BlockSpec((BLOCK, h), lambda i: (i, 0)),
            pl.BlockSpec((NEXP, h), lambda i: (0, 0)),
        ],
        out_specs=[
            pl.BlockSpec((TOPK, BLOCK), lambda i: (0, i)),
            pl.BlockSpec((TOPK, BLOCK), lambda i: (0, i)),
        ],
        out_shape=[
            jax.ShapeDtypeStruct((TOPK, n), jnp.int32),
            jax.ShapeDtypeStruct((TOPK, n), jnp.float32),
        ],
        compiler_params=pltpu.CompilerParams(
            dimension_semantics=("arbitrary",),
        ),
    )(x, weight)


def _make_sc_probe(nsc, h):
    mesh = plsc.VectorSubcoreMesh(core_axis_name="c", subcore_axis_name="s")
    ntok = nsc // 32
    nch = ntok // 16

    @functools.partial(
        pl.kernel,
        mesh=mesh,
        out_type=jax.ShapeDtypeStruct((32, 16), jnp.float32),
        scratch_types=[
            pltpu.VMEM((16, h), jnp.float32),
            pltpu.SemaphoreType.DMA,
        ],
    )
    def sc_probe(x_hbm, out_hbm, xbuf, sem):
        wid = lax.axis_index("s") * 2 + lax.axis_index("c")
        base = wid * ntok

        def body(c, acc):
            pltpu.async_copy(
                x_hbm.at[pl.ds(base + c * 16, 16), :], xbuf, sem
            ).wait()

            def ib(i, a):
                return a + xbuf[0, pl.ds(i * 16, 16)]

            return lax.fori_loop(0, h // 16, ib, acc)

        acc = lax.fori_loop(0, nch, body, jnp.zeros((16,), jnp.float32))
        xbuf[0, pl.ds(0, 16)] = acc
        pltpu.sync_copy(xbuf.at[0, pl.ds(0, 16)], out_hbm.at[wid])

    return sc_probe


@jax.jit
def kernel(hidden_states, weight):
    bsz, seq_len, h = hidden_states.shape
    n = bsz * seq_len
    x = hidden_states.reshape(n, h)

    idx_t, wgt_t = _tc_gate(x, weight)
    dummy = _make_sc_probe(SC_TOKENS, h)(x)
    return idx_t.T, wgt_t.T + 0.0 * jnp.sum(dummy)


# speedup vs baseline: 1.0898x; 1.0898x over previous
import functools

import jax
import jax.numpy as jnp
from jax import lax
from jax.experimental import pallas as pl
from jax.experimental.pallas import tpu as pltpu
from jax.experimental.pallas import tpu_sc as plsc

TOPK = 2
NEXP = 8
BLOCK = 1024
NWORK = 32
NLANE = 16


def _matmul_kernel(x_ref, w_ref, logits_ref):
    logits_ref[...] = jax.lax.dot_general(
        w_ref[...], x_ref[...], (((1,), (1,)), ((), ())),
        preferred_element_type=jnp.float32,
    )


def _tc_logits(x, weight):
    n, h = x.shape
    return pl.pallas_call(
        _matmul_kernel,
        grid=(n // BLOCK,),
        in_specs=[
            pl.BlockSpec((BLOCK, h), lambda i: (i, 0)),
            pl.BlockSpec((NEXP, h), lambda i: (0, 0)),
        ],
        out_specs=pl.BlockSpec((NEXP, BLOCK), lambda i: (0, i)),
        out_shape=jax.ShapeDtypeStruct((NEXP, n), jnp.float32),
        compiler_params=pltpu.CompilerParams(
            dimension_semantics=("arbitrary",),
        ),
    )(x, weight)


def _make_sc_router(n):
    ntok = n // NWORK
    ngrp = ntok // NLANE
    mesh = plsc.VectorSubcoreMesh(core_axis_name="c", subcore_axis_name="s")

    @functools.partial(
        pl.kernel,
        mesh=mesh,
        out_type=[
            jax.ShapeDtypeStruct((TOPK, n), jnp.int32),
            jax.ShapeDtypeStruct((TOPK, n), jnp.float32),
        ],
        scratch_types=[
            pltpu.VMEM((NEXP, ntok), jnp.float32),
            pltpu.VMEM((TOPK, ntok), jnp.int32),
            pltpu.VMEM((TOPK, ntok), jnp.float32),
            pltpu.SemaphoreType.DMA,
        ],
    )
    def sc_router(logits_hbm, idx_hbm, wgt_hbm, lbuf, ibuf, wbuf, sem):
        wid = lax.axis_index("s") * 2 + lax.axis_index("c")
        base = wid * ntok
        pltpu.async_copy(
            logits_hbm.at[:, pl.ds(base, ntok)], lbuf, sem
        ).wait()

        neg_inf = jnp.full((NLANE,), -jnp.inf, jnp.float32)

        def body(g, carry):
            off = g * NLANE
            ls = [lbuf[e, pl.ds(off, NLANE)] for e in range(NEXP)]
            m1 = ls[0]
            for e in range(1, NEXP):
                m1 = jnp.maximum(m1, ls[e])
            idx1 = jnp.full((NLANE,), NEXP, jnp.int32)
            for e in range(NEXP - 1, -1, -1):
                idx1 = jnp.where(ls[e] == m1, e, idx1)
            m2 = neg_inf
            for e in range(NEXP):
                m2 = jnp.maximum(m2, jnp.where(idx1 == e, neg_inf, ls[e]))
            idx2 = jnp.full((NLANE,), NEXP, jnp.int32)
            for e in range(NEXP - 1, -1, -1):
                idx2 = jnp.where((ls[e] == m2) & (idx1 != e), e, idx2)
            z = jnp.zeros((NLANE,), jnp.float32)
            for e in range(NEXP):
                z = z + jnp.exp(ls[e] - m1)
            w1 = 1.0 / z
            w2 = jnp.exp(m2 - m1) / z
            ibuf[0, pl.ds(off, NLANE)] = idx1
            ibuf[1, pl.ds(off, NLANE)] = idx2
            wbuf[0, pl.ds(off, NLANE)] = w1
            wbuf[1, pl.ds(off, NLANE)] = w2
            return carry

        lax.fori_loop(0, ngrp, body, 0)
        pltpu.sync_copy(ibuf, idx_hbm.at[:, pl.ds(base, ntok)])
        pltpu.sync_copy(wbuf, wgt_hbm.at[:, pl.ds(base, ntok)])

    return sc_router


@jax.jit
def kernel(hidden_states, weight):
    bsz, seq_len, h = hidden_states.shape
    n = bsz * seq_len
    x = hidden_states.reshape(n, h)

    logits = _tc_logits(x, weight)
    idx_t, wgt_t = _make_sc_router(n)(logits)
    return idx_t.T, wgt_t.T


# speedup vs baseline: 1.6005x; 1.4686x over previous
import jax
import jax.numpy as jnp
from jax.experimental import pallas as pl
from jax.experimental.pallas import tpu as pltpu

TOPK = 2
NEXP = 8
BLOCK = 1024


def _gate_kernel(x_ref, w_ref, idx_ref, wgt_ref):
    x = x_ref[...]
    w = w_ref[...]
    logits = jax.lax.dot_general(
        w, x, (((1,), (1,)), ((), ())), preferred_element_type=jnp.float32
    )
    T = logits.shape[1]
    iota = jax.lax.broadcasted_iota(jnp.int32, (NEXP, T), 0)

    m1 = jnp.max(logits, axis=0, keepdims=True)
    idx1 = jnp.min(jnp.where(logits == m1, iota, NEXP), axis=0, keepdims=True)

    masked = jnp.where(iota == idx1, -jnp.inf, logits)
    m2 = jnp.max(masked, axis=0, keepdims=True)
    idx2 = jnp.min(jnp.where(masked == m2, iota, NEXP), axis=0, keepdims=True)

    z = jnp.sum(jnp.exp(logits - m1), axis=0, keepdims=True)
    w1 = 1.0 / z
    w2 = jnp.exp(m2 - m1) / z

    idx_ref[...] = jnp.concatenate([idx1, idx2], axis=0)
    wgt_ref[...] = jnp.concatenate([w1, w2], axis=0)


@jax.jit
def kernel(hidden_states, weight):
    bsz, seq_len, h = hidden_states.shape
    n = bsz * seq_len
    x = hidden_states.reshape(n, h)

    grid = (n // BLOCK,)
    idx_t, wgt_t = pl.pallas_call(
        _gate_kernel,
        grid=grid,
        in_specs=[
            pl.BlockSpec((BLOCK, h), lambda i: (i, 0)),
            pl.BlockSpec((NEXP, h), lambda i: (0, 0)),
        ],
        out_specs=[
            pl.BlockSpec((TOPK, BLOCK), lambda i: (0, i)),
            pl.BlockSpec((TOPK, BLOCK), lambda i: (0, i)),
        ],
        out_shape=[
            jax.ShapeDtypeStruct((TOPK, n), jnp.int32),
            jax.ShapeDtypeStruct((TOPK, n), jnp.float32),
        ],
        compiler_params=pltpu.CompilerParams(
            dimension_semantics=("parallel",),
        ),
    )(x, weight)
    return idx_t.T, wgt_t.T
